# baseline (speedup 1.0000x reference)
import functools
import numpy as np
import jax
import jax.numpy as jnp
from jax import lax
from jax.experimental import pallas as pl
from jax.experimental.pallas import tpu as pltpu
from jax.experimental.pallas import tpu_sc as plsc

H = 128
ED = 16
NH = 4
DH = H // NH
BN_EPS = 1e-5
BNS = float(1.0 / np.sqrt(1.0 + BN_EPS))


_SC_MESH = dict(core_axis_name="c", subcore_axis_name="s")
_NW = 32


def _wid():
    return lax.axis_index("s") * 2 + lax.axis_index("c")


def _sc_gather_sum(tables, idxs):
    K = len(tables)
    D = tables[0].shape[1]
    B = idxs[0].shape[0]
    assert B % 128 == 0 and D % 16 == 0
    NT = B // 128
    NB = 3

    @functools.partial(
        pl.kernel,
        mesh=plsc.VectorSubcoreMesh(**_SC_MESH),
        out_type=jax.ShapeDtypeStruct((B, D), jnp.float32),
        scratch_types=[
            pltpu.VMEM((NB * K, 128), jnp.int32),
            pltpu.VMEM((NB * K, 128, D), jnp.float32),
            pltpu.SemaphoreType.DMA((NB * K,)),
        ],
    )
    def k(*refs):
        tab_hbm = refs[:K]
        idx_hbm = refs[K:2 * K]
        out_hbm = refs[2 * K]
        idx_v, rows_v, sem = refs[2 * K + 1:]
        w = _wid()
        n = (NT - w + _NW - 1) // _NW

        def fire(i, b):
            t = w + i * _NW
            for s in range(K):
                j = b * K + s
                pltpu.sync_copy(idx_hbm[s].at[pl.ds(t * 128, 128)], idx_v.at[j])
                pltpu.async_copy(tab_hbm[s].at[idx_v.at[j]], rows_v.at[j],
                                 sem.at[j])

        for b in range(NB):
            @pl.when(b < n)
            def _prime():
                fire(b, b)

        def outer(io, carry):
            for b in range(NB):
                i = io * NB + b

                @pl.when(i < n)
                def _step():
                    t = w + i * _NW
                    for s in range(K):
                        j = b * K + s
                        pltpu.make_async_copy(
                            tab_hbm[s].at[pl.ds(0, 128)], rows_v.at[j],
                            sem.at[j]).wait()
                    if K == 2:
                        r0, r1 = rows_v.at[b * K], rows_v.at[b * K + 1]

                        def add_row(r, c2):
                            for l in range(D // 16):
                                sl = pl.ds(l * 16, 16)
                                r0[r, sl] = r0[r, sl] + r1[r, sl]
                            return c2

                        lax.fori_loop(0, 128, add_row, 0)
                    pltpu.sync_copy(rows_v.at[b * K], out_hbm.at[pl.ds(t * 128, 128)])

                    @pl.when(i + NB < n)
                    def _next():
                        fire(i + NB, b)
            return carry

        lax.fori_loop(0, (n + NB - 1) // NB, outer, 0)

    return k(*tables, *idxs)


def _sc_gather(table, idx):
    return _sc_gather_sum([table], [idx])


def _ceil(a, b):
    return -(-a // b)


def _sc_segsum(rows, ids, nseg):
    E, D = rows.shape
    assert E % 128 == 0 and D % 16 == 0
    NC = 2 * _ceil(nseg, 13000)
    C = 128 * _ceil(nseg, NC * 128)
    C16 = C // 16
    npad = NC * C

    starts = jnp.arange(NC, dtype=jnp.int32) * C
    lo = jnp.searchsorted(ids, starts).astype(jnp.int32)
    hi = jnp.searchsorted(ids, starts + C).astype(jnp.int32)
    t0 = lo // 128
    nt = (hi + 127) // 128 - t0
    t0 = jnp.concatenate([t0, jnp.zeros((32 - NC,), jnp.int32)])
    nt = jnp.concatenate([nt, jnp.zeros((32 - NC,), jnp.int32)])

    @functools.partial(
        pl.kernel,
        mesh=plsc.VectorSubcoreMesh(**_SC_MESH),
        out_type=jax.ShapeDtypeStruct((npad, D), jnp.float32),
        scratch_types=[
            pltpu.VMEM((128,), jnp.int32),
            pltpu.VMEM((128, D), jnp.float32),
            pltpu.VMEM((64,), jnp.int32),
            pltpu.VMEM((C16, D), jnp.float32),
            pltpu.VMEM_SHARED((C + 16, D), jnp.float32),
        ],
    )
    def k(rows_hbm, ids_hbm, t0_hbm, nt_hbm, out_hbm,
          idx_v, rows_v, bounds_v, zrow_v, acc_sh):
        cid = lax.axis_index("c")
        sid = lax.axis_index("s")
        pltpu.sync_copy(t0_hbm, bounds_v.at[pl.ds(0, 32)])
        pltpu.sync_copy(nt_hbm, bounds_v.at[pl.ds(32, 32)])

        def zbody(r, carry):
            for l in range(D // 16):
                zrow_v[r, pl.ds(l * 16, 16)] = jnp.zeros((16,), jnp.float32)
            return carry

        lax.fori_loop(0, C16, zbody, 0)

        def _extract(off, ci):
            seg = bounds_v[pl.ds(off + 16 * (ci // 8), 16)]
            k0 = (2 * ci) % 16
            return jnp.where(cid == 0, seg[k0], seg[k0 + 1])

        for ci in range(NC // 2):
            c = 2 * ci + cid
            t0c = _extract(0, ci)
            ntc = _extract(32, ci)
            cbase = c * C
            sl_acc = pl.ds(pl.multiple_of(sid * C16, 8), C16)
            pltpu.sync_copy(zrow_v, acc_sh.at[sl_acc])
            plsc.subcore_barrier()

            def body(j2, carry):
                tt = t0c + sid + 16 * j2
                e0 = pl.multiple_of(tt * 128, 128)
                pltpu.sync_copy(ids_hbm.at[pl.ds(e0, 128)], idx_v)
                pltpu.sync_copy(rows_hbm.at[pl.ds(e0, 128)], rows_v)
                for l in range(8):
                    sl = pl.ds(l * 16, 16)
                    v = idx_v[sl] - cbase
                    ok = (v >= 0) & (v < C)
                    idx_v[sl] = jnp.where(ok, v, C)
                pltpu.sync_copy(rows_v, acc_sh.at[idx_v], add=True)
                return carry

            lax.fori_loop(0, (ntc - sid + 15) // 16, body, 0)
            plsc.subcore_barrier()
            pltpu.sync_copy(acc_sh.at[sl_acc],
                            out_hbm.at[pl.ds(pl.multiple_of(cbase + sid * C16, 8),
                                             C16)])

    out = k(rows, ids, t0, nt)
    return out[:nseg] if npad != nseg else out


def _sc_gather2sum(t1, idx1, t2, idx2):
    return _sc_gather_sum([t1, t2], [idx1, idx2])


def _final_body(x_ref, aggr_ref, g_ref, wskip_ref, w1_ref, w2_ref, c_ref, out_ref):
    x = x_ref[...]
    skip = jnp.dot(x, wskip_ref[...], preferred_element_type=jnp.float32)
    u = c_ref[5:6, :] * x + aggr_ref[...]
    t = jnp.maximum(jnp.dot(u, w1_ref[...], preferred_element_type=jnp.float32)
                    + c_ref[1:2, :], 0.0)
    h1 = jnp.dot(t, w2_ref[...], preferred_element_type=jnp.float32) + c_ref[2:3, :]
    h1 = h1 * c_ref[3:4, :] + c_ref[4:5, :]
    out_ref[...] = jnp.maximum(skip + c_ref[0:1, :] + h1 + g_ref[...], 0.0)


def _final_combine(x, aggr, g, p):
    F = x.shape[0]
    BF = 2000
    grid = (F // BF,)
    consts = jnp.stack([
        p['skip_b'], p['loc_b1'], p['loc_b2'],
        p['loc_bn_g'] * BNS, p['loc_bn_b'],
        jnp.full((H,), 1.0 + p['loc_eps'], jnp.float32),
        jnp.zeros((H,), jnp.float32), jnp.zeros((H,), jnp.float32),
    ])
    row_spec = pl.BlockSpec((BF, H), lambda i: (i, 0))
    w_spec = pl.BlockSpec((H, H), lambda i: (0, 0))
    return pl.pallas_call(
        _final_body,
        grid=grid,
        in_specs=[row_spec, row_spec, row_spec, w_spec, w_spec, w_spec,
                  pl.BlockSpec((8, H), lambda i: (0, 0))],
        out_specs=row_spec,
        out_shape=jax.ShapeDtypeStruct((F, H), jnp.float32),
    )(x, aggr, g, p['skip_W'].T, p['loc_W1'].T, p['loc_W2'].T, consts)


def _bn(x, g, b):
    return x * BNS * g + b


def _mlp(x, W1, b1, W2, b2):
    return jax.nn.relu(x @ W1.T + b1) @ W2.T + b2


def kernel(h_flat, intra_ei, ea_flat, valid, node_ids, N_total, edge_index,
           edge_attr, sub_batch, S, root_flat_idx, m, params):
    p = params
    F = h_flat.shape[0]
    S_static = root_flat_idx.shape[0]
    m_static = 4
    N_static = S_static // m_static

    src, dst = intra_ei[0], intra_ei[1]
    perm = jnp.argsort(dst)
    src_s, dst_s = src[perm], dst[perm]
    e = ea_flat[perm] @ p['loc_edge_W'].T + p['loc_edge_b']
    msg = jax.nn.relu(h_flat[src_s] + e)
    aggr = _sc_segsum(msg, dst_s, F)

    root_ids = node_ids[root_flat_idx]
    order = jnp.argsort(root_ids, stable=True)
    h_2d = h_flat[root_flat_idx][order].reshape(N_static, m_static, H)
    qkv = h_2d @ p['attn_in_W'].T + p['attn_in_b']
    q, k, v = jnp.split(qkv, 3, axis=-1)
    hd = lambda t: t.reshape(N_static, m_static, NH, DH).transpose(0, 2, 1, 3)
    q, k, v = hd(q), hd(k), hd(v)
    s = (q @ k.transpose(0, 1, 3, 2)) / np.sqrt(DH)
    a = jax.nn.softmax(s, axis=-1)
    o2 = (a @ v).transpose(0, 2, 1, 3).reshape(N_static, m_static, H)
    h_attn = o2 @ p['attn_out_W'].T + p['attn_out_b'] + h_2d
    h_attn_node = _bn(h_attn.mean(axis=1), p['attn_bn_g'], p['attn_bn_b'])

    src2, dst2 = edge_index[0], edge_index[1]
    e2 = edge_attr @ p['glob_edge_W'].T + p['glob_edge_b']
    msg2 = jax.nn.relu(h_attn_node[src2] + e2)
    aggr2 = jnp.zeros_like(h_attn_node).at[dst2].add(msg2)
    h2 = _mlp((1.0 + p['glob_eps']) * h_attn_node + aggr2,
              p['glob_W1'], p['glob_b1'], p['glob_W2'], p['glob_b2'])
    h2 = _bn(h2, p['glob_bn_g'], p['glob_bn_b'])

    sums = jax.ops.segment_sum(h_flat, sub_batch, num_segments=S_static)
    cnts = jax.ops.segment_sum(jnp.ones((F,), jnp.float32), sub_batch,
                               num_segments=S_static)
    h_sub = sums / jnp.maximum(cnts, 1.0)[:, None]
    h_sub = _bn(_mlp(h_sub, p['sub_W1'], p['sub_b1'], p['sub_W2'], p['sub_b2']),
                p['sub_bn_g'], p['sub_bn_b'])

    g = _sc_gather2sum(h_attn_node + h2, node_ids, h_sub, sub_batch)
    return _final_combine(h_flat, aggr, g, p)


# speedup vs baseline: 1.6043x; 1.6043x over previous
import functools
import numpy as np
import jax
import jax.numpy as jnp
from jax import lax
from jax.experimental import pallas as pl
from jax.experimental.pallas import tpu as pltpu
from jax.experimental.pallas import tpu_sc as plsc

H = 128
ED = 16
NH = 4
DH = H // NH
BN_EPS = 1e-5
BNS = float(1.0 / np.sqrt(1.0 + BN_EPS))


_SC_MESH = dict(core_axis_name="c", subcore_axis_name="s")
_NW = 32


def _wid():
    return lax.axis_index("s") * 2 + lax.axis_index("c")


def _sc_gather_sum(tables, idxs):
    K = len(tables)
    D = tables[0].shape[1]
    B = idxs[0].shape[0]
    assert B % 128 == 0 and D % 16 == 0
    NT = B // 128
    NB = 3

    @functools.partial(
        pl.kernel,
        mesh=plsc.VectorSubcoreMesh(**_SC_MESH),
        out_type=jax.ShapeDtypeStruct((B, D), jnp.float32),
        scratch_types=[
            pltpu.VMEM((NB * K, 128), jnp.int32),
            pltpu.VMEM((NB * K, 128, D), jnp.float32),
            pltpu.SemaphoreType.DMA((NB * K,)),
        ],
    )
    def k(*refs):
        tab_hbm = refs[:K]
        idx_hbm = refs[K:2 * K]
        out_hbm = refs[2 * K]
        idx_v, rows_v, sem = refs[2 * K + 1:]
        w = _wid()
        n = (NT - w + _NW - 1) // _NW

        def fire(i, b):
            t = w + i * _NW
            for s in range(K):
                j = b * K + s
                pltpu.sync_copy(idx_hbm[s].at[pl.ds(t * 128, 128)], idx_v.at[j])
                pltpu.async_copy(tab_hbm[s].at[idx_v.at[j]], rows_v.at[j],
                                 sem.at[j])

        for b in range(NB):
            @pl.when(b < n)
            def _prime():
                fire(b, b)

        def outer(io, carry):
            for b in range(NB):
                i = io * NB + b

                @pl.when(i < n)
                def _step():
                    t = w + i * _NW
                    for s in range(K):
                        j = b * K + s
                        pltpu.make_async_copy(
                            tab_hbm[s].at[pl.ds(0, 128)], rows_v.at[j],
                            sem.at[j]).wait()
                    if K == 2:
                        r0, r1 = rows_v.at[b * K], rows_v.at[b * K + 1]

                        def add_row(r, c2):
                            for l in range(D // 16):
                                sl = pl.ds(l * 16, 16)
                                r0[r, sl] = r0[r, sl] + r1[r, sl]
                            return c2

                        lax.fori_loop(0, 128, add_row, 0)
                    pltpu.sync_copy(rows_v.at[b * K], out_hbm.at[pl.ds(t * 128, 128)])

                    @pl.when(i + NB < n)
                    def _next():
                        fire(i + NB, b)
            return carry

        lax.fori_loop(0, (n + NB - 1) // NB, outer, 0)

    return k(*tables, *idxs)


def _sc_gather(table, idx):
    return _sc_gather_sum([table], [idx])


def _sc_gather2sum(t1, idx1, t2, idx2):
    return _sc_gather_sum([t1, t2], [idx1, idx2])


def _final_body(x_ref, aggr_ref, g_ref, wskip_ref, w1_ref, w2_ref, c_ref, out_ref):
    x = x_ref[...]
    skip = jnp.dot(x, wskip_ref[...], preferred_element_type=jnp.float32)
    u = c_ref[5:6, :] * x + aggr_ref[...]
    t = jnp.maximum(jnp.dot(u, w1_ref[...], preferred_element_type=jnp.float32)
                    + c_ref[1:2, :], 0.0)
    h1 = jnp.dot(t, w2_ref[...], preferred_element_type=jnp.float32) + c_ref[2:3, :]
    h1 = h1 * c_ref[3:4, :] + c_ref[4:5, :]
    out_ref[...] = jnp.maximum(skip + c_ref[0:1, :] + h1 + g_ref[...], 0.0)


def _final_combine(x, aggr, g, p):
    F = x.shape[0]
    BF = 2000
    grid = (F // BF,)
    consts = jnp.stack([
        p['skip_b'], p['loc_b1'], p['loc_b2'],
        p['loc_bn_g'] * BNS, p['loc_bn_b'],
        jnp.full((H,), 1.0 + p['loc_eps'], jnp.float32),
        jnp.zeros((H,), jnp.float32), jnp.zeros((H,), jnp.float32),
    ])
    row_spec = pl.BlockSpec((BF, H), lambda i: (i, 0))
    w_spec = pl.BlockSpec((H, H), lambda i: (0, 0))
    return pl.pallas_call(
        _final_body,
        grid=grid,
        in_specs=[row_spec, row_spec, row_spec, w_spec, w_spec, w_spec,
                  pl.BlockSpec((8, H), lambda i: (0, 0))],
        out_specs=row_spec,
        out_shape=jax.ShapeDtypeStruct((F, H), jnp.float32),
    )(x, aggr, g, p['skip_W'].T, p['loc_W1'].T, p['loc_W2'].T, consts)


def _bn(x, g, b):
    return x * BNS * g + b


def _mlp(x, W1, b1, W2, b2):
    return jax.nn.relu(x @ W1.T + b1) @ W2.T + b2


def kernel(h_flat, intra_ei, ea_flat, valid, node_ids, N_total, edge_index,
           edge_attr, sub_batch, S, root_flat_idx, m, params):
    p = params
    F = h_flat.shape[0]
    S_static = root_flat_idx.shape[0]
    m_static = 4
    N_static = S_static // m_static

    src, dst = intra_ei[0], intra_ei[1]
    e = ea_flat @ p['loc_edge_W'].T + p['loc_edge_b']
    msg = jax.nn.relu(h_flat[src] + e)
    aggr = jnp.zeros_like(h_flat).at[dst].add(msg)

    root_ids = node_ids[root_flat_idx]
    order = jnp.argsort(root_ids, stable=True)
    h_2d = h_flat[root_flat_idx][order].reshape(N_static, m_static, H)
    qkv = h_2d @ p['attn_in_W'].T + p['attn_in_b']
    q, k, v = jnp.split(qkv, 3, axis=-1)
    hd = lambda t: t.reshape(N_static, m_static, NH, DH).transpose(0, 2, 1, 3)
    q, k, v = hd(q), hd(k), hd(v)
    s = (q @ k.transpose(0, 1, 3, 2)) / np.sqrt(DH)
    a = jax.nn.softmax(s, axis=-1)
    o2 = (a @ v).transpose(0, 2, 1, 3).reshape(N_static, m_static, H)
    h_attn = o2 @ p['attn_out_W'].T + p['attn_out_b'] + h_2d
    h_attn_node = _bn(h_attn.mean(axis=1), p['attn_bn_g'], p['attn_bn_b'])

    src2, dst2 = edge_index[0], edge_index[1]
    e2 = edge_attr @ p['glob_edge_W'].T + p['glob_edge_b']
    msg2 = jax.nn.relu(h_attn_node[src2] + e2)
    aggr2 = jnp.zeros_like(h_attn_node).at[dst2].add(msg2)
    h2 = _mlp((1.0 + p['glob_eps']) * h_attn_node + aggr2,
              p['glob_W1'], p['glob_b1'], p['glob_W2'], p['glob_b2'])
    h2 = _bn(h2, p['glob_bn_g'], p['glob_bn_b'])

    sums = jax.ops.segment_sum(h_flat, sub_batch, num_segments=S_static)
    cnts = jax.ops.segment_sum(jnp.ones((F,), jnp.float32), sub_batch,
                               num_segments=S_static)
    h_sub = sums / jnp.maximum(cnts, 1.0)[:, None]
    h_sub = _bn(_mlp(h_sub, p['sub_W1'], p['sub_b1'], p['sub_W2'], p['sub_b2']),
                p['sub_bn_g'], p['sub_bn_b'])

    g = _sc_gather2sum(h_attn_node + h2, node_ids, h_sub, sub_batch)
    return _final_combine(h_flat, aggr, g, p)
